# baseline (speedup 1.0000x reference)
import jax
import jax.numpy as jnp
from jax.experimental import pallas as pl
from jax.experimental.pallas import tpu as pltpu

_B, _T, _C, _H, _W = 8, 16, 64, 64, 64
_T2 = _T // 2
_G = 16
_NCHUNK = _B * _T
_NS = 22
_D = 11
_U = 8


def _body(x_ref, v_ref, o_ref, buf, in_sems, out_sems):
    even = (jax.lax.broadcasted_iota(jnp.int32, (_G, 16, 64), 2) % 2) == 0
    q128 = jax.lax.broadcasted_iota(jnp.int32, (128, 64), 0)
    l64 = jax.lax.broadcasted_iota(jnp.int32, (128, 64), 1)

    def in_copy(j):
        s = jax.lax.rem(j, _NS)
        b = jax.lax.div(j, _T)
        t = jax.lax.rem(j, _T)
        return pltpu.make_async_copy(x_ref.at[b, t], buf.at[s], in_sems.at[s])

    def out_copy(j):
        s = jax.lax.rem(j, _NS)
        b = jax.lax.div(j, _T)
        t = jax.lax.rem(j, _T)
        return pltpu.make_async_copy(buf.at[s], o_ref.at[b, t], out_sems.at[s])

    for m in range(_D):
        in_copy(m).start(priority=m % 2)

    def step(g, c):
        for p in range(_U):
            j = g * _U + p

            @pl.when(j >= _NS - _D)
            def _():
                out_copy(j - (_NS - _D)).wait()

            @pl.when(j + _D < _NCHUNK)
            def _(p=p, j=j):
                in_copy(j + _D).start(priority=(p + _D) % 2)

            in_copy(j).wait()
            if p % 2 == 0:
                sl = jax.lax.rem(j, _NS)
                b = jax.lax.div(j, _T)
                t2 = jax.lax.div(jax.lax.rem(j, _T), 2)
                vmat = v_ref[b, t2].reshape(_G * 16, 128)
                for s in range(4):
                    es = jnp.where(
                        (l64 % 2 == 0) & (q128 == 32 * s + l64 // 2),
                        1.0, 0.0).astype(jnp.float32)
                    ws = jax.lax.dot_general(
                        vmat, es, (((1,), (0,)), ((), ())),
                        preferred_element_type=jnp.float32,
                        precision=jax.lax.Precision.HIGHEST)
                    ws = ws.reshape(_G, 16, 64)
                    xs = buf[sl, pl.ds(0, _G, 4), pl.ds(s, 16, 4), :]
                    buf[sl, pl.ds(0, _G, 4), pl.ds(s, 16, 4), :] = (
                        jnp.where(even, ws, xs))
            out_copy(j).start(priority=p % 2)
        return c

    jax.lax.fori_loop(0, _NCHUNK // _U, step, 0, unroll=False)

    for m in range(_NCHUNK - (_NS - _D), _NCHUNK):
        out_copy(m).wait()


def kernel(x_aug, values):
    v4 = values.reshape(_B, _T2, _G, 16, 128)
    out = pl.pallas_call(
        _body,
        in_specs=[
            pl.BlockSpec(memory_space=pl.ANY),
            pl.BlockSpec(memory_space=pltpu.VMEM),
        ],
        out_specs=pl.BlockSpec(memory_space=pl.ANY),
        out_shape=jax.ShapeDtypeStruct((_B, _T, _C, _H, _W), jnp.float32),
        scratch_shapes=[
            pltpu.VMEM((_NS, _C, _H, _W), jnp.float32),
            pltpu.SemaphoreType.DMA((_NS,)),
            pltpu.SemaphoreType.DMA((_NS,)),
        ],
    )(x_aug, v4)
    return out


# speedup vs baseline: 1.0026x; 1.0026x over previous
import jax
import jax.numpy as jnp
from jax.experimental import pallas as pl
from jax.experimental.pallas import tpu as pltpu

_B, _T, _C, _H, _W = 8, 16, 64, 64, 64
_T2 = _T // 2
_G = 16
_NCHUNK = _B * _T2
_NS = 11
_D = 6
_U = 4


def _body(x_ref, v_ref, o_ref, buf, in_sems, out_sems):
    even = (jax.lax.broadcasted_iota(jnp.int32, (_G, 16, 64), 2) % 2) == 0
    q128 = jax.lax.broadcasted_iota(jnp.int32, (128, 64), 0)
    l64 = jax.lax.broadcasted_iota(jnp.int32, (128, 64), 1)

    def in_copy(j):
        s = jax.lax.rem(j, _NS)
        b = jax.lax.div(j, _T2)
        t2 = jax.lax.rem(j, _T2)
        return pltpu.make_async_copy(
            x_ref.at[b, pl.ds(2 * t2, 2)], buf.at[s], in_sems.at[s])

    def out_copy(j):
        s = jax.lax.rem(j, _NS)
        b = jax.lax.div(j, _T2)
        t2 = jax.lax.rem(j, _T2)
        return pltpu.make_async_copy(
            buf.at[s], o_ref.at[b, pl.ds(2 * t2, 2)], out_sems.at[s])

    for m in range(_D):
        in_copy(m).start(priority=m % 2)

    def step(g, c):
        for p in range(_U):
            j = g * _U + p

            @pl.when(j >= _NS - _D)
            def _():
                out_copy(j - (_NS - _D)).wait()

            @pl.when(j + _D < _NCHUNK)
            def _(p=p, j=j):
                in_copy(j + _D).start(priority=(p + 1) % 2)

            in_copy(j).wait()
            sl = jax.lax.rem(j, _NS)
            b = jax.lax.div(j, _T2)
            t2 = jax.lax.rem(j, _T2)
            vmat = v_ref[b, t2].reshape(_G * 16, 128)
            for s in range(4):
                es = jnp.where(
                    (l64 % 2 == 0) & (q128 == 32 * s + l64 // 2),
                    1.0, 0.0).astype(jnp.float32)
                ws = jax.lax.dot_general(
                    vmat, es, (((1,), (0,)), ((), ())),
                    preferred_element_type=jnp.float32,
                    precision=jax.lax.Precision.HIGHEST)
                ws = ws.reshape(_G, 16, 64)
                xs = buf[sl, 0, pl.ds(0, _G, 4), pl.ds(s, 16, 4), :]
                buf[sl, 0, pl.ds(0, _G, 4), pl.ds(s, 16, 4), :] = (
                    jnp.where(even, ws, xs))
            out_copy(j).start(priority=p % 2)
        return c

    jax.lax.fori_loop(0, _NCHUNK // _U, step, 0, unroll=False)

    for m in range(_NCHUNK - (_NS - _D), _NCHUNK):
        out_copy(m).wait()


def kernel(x_aug, values):
    v4 = values.reshape(_B, _T2, _G, 16, 128)
    out = pl.pallas_call(
        _body,
        in_specs=[
            pl.BlockSpec(memory_space=pl.ANY),
            pl.BlockSpec(memory_space=pltpu.VMEM),
        ],
        out_specs=pl.BlockSpec(memory_space=pl.ANY),
        out_shape=jax.ShapeDtypeStruct((_B, _T, _C, _H, _W), jnp.float32),
        scratch_shapes=[
            pltpu.VMEM((_NS, 2, _C, _H, _W), jnp.float32),
            pltpu.SemaphoreType.DMA((_NS,)),
            pltpu.SemaphoreType.DMA((_NS,)),
        ],
    )(x_aug, v4)
    return out
